# baseline (speedup 1.0000x reference)
import functools

import jax
import jax.numpy as jnp
from jax import lax
from jax.experimental import pallas as pl
from jax.experimental.pallas import tpu as pltpu
from jax.experimental.pallas import tpu_sc as plsc

_NUM_CORES = 2
_NUM_SUBCORES = 16
_NUM_WORKERS = _NUM_CORES * _NUM_SUBCORES
_LANES = 16
_NBUF = 7
_CHUNK = 8


def _emb_lookup(labels_flat, wte, wpe, batch, seq_len):
    n = labels_flat.shape[0]
    _, d = wte.shape
    pos_per_w = seq_len // _NUM_WORKERS
    per_w = n // _NUM_WORKERS
    n_chunks = per_w // _CHUNK
    chunks_per_seq = pos_per_w // _CHUNK

    mesh = plsc.VectorSubcoreMesh(core_axis_name="c", subcore_axis_name="s")

    @functools.partial(
        pl.kernel,
        mesh=mesh,
        out_type=jax.ShapeDtypeStruct((batch, seq_len, d), jnp.float32),
        scratch_types=(
            [pltpu.VMEM((per_w,), jnp.int32),
             pltpu.VMEM((pos_per_w, d), jnp.float32)]
            + [pltpu.VMEM((_CHUNK, d), jnp.float32) for _ in range(_NBUF)]
            + [pltpu.SemaphoreType.DMA for _ in range(2 * _NBUF)]
        ),
    )
    def emb_kernel(wte_hbm, idx_hbm, wpe_hbm, out_hbm, idx_v, wpe_v, *bufs_and_sems):
        rows = bufs_and_sems[0:_NBUF]
        isems = bufs_and_sems[_NBUF:2 * _NBUF]
        osems = bufs_and_sems[2 * _NBUF:3 * _NBUF]

        wid = lax.axis_index("s") * _NUM_CORES + lax.axis_index("c")
        pos_base = wid * pos_per_w

        wpe_load = pltpu.make_async_copy(
            wpe_hbm.at[pl.ds(pos_base, pos_per_w)], wpe_v, osems[0]
        )
        wpe_load.start()
        idx_loads = [
            pltpu.make_async_copy(
                idx_hbm.at[pl.ds(b_idx * seq_len + pos_base, pos_per_w)],
                idx_v.at[pl.ds(b_idx * pos_per_w, pos_per_w)],
                isems[0],
            )
            for b_idx in range(batch)
        ]
        for cp in idx_loads:
            cp.start()
        for cp in idx_loads:
            cp.wait()

        def gather(c, b):
            return pltpu.make_async_copy(
                wte_hbm.at[idx_v.at[pl.ds(c * _CHUNK, _CHUNK)]], rows[b], isems[b]
            )

        def out_copy(c, b):
            b_idx, cc = divmod(c, chunks_per_seq)
            return pltpu.make_async_copy(
                rows[b],
                out_hbm.at[b_idx, pl.ds(pos_base + cc * _CHUNK, _CHUNK)],
                osems[b],
            )

        for c0 in range(_NBUF - 1):
            gather(c0, c0).start()
        for c in range(n_chunks):
            b = c % _NBUF
            if c == 0:
                wpe_load.wait()
            gather(c, b).wait()
            wrow_base = (c % chunks_per_seq) * _CHUNK

            buf = rows[b]

            @plsc.parallel_loop(0, d, step=_LANES)
            def _lane_loop(j, _buf=buf, _wb=wrow_base):
                for r in range(_CHUNK):
                    plsc.addupdate(
                        _buf.at[r, pl.ds(j, _LANES)],
                        wpe_v[_wb + r, pl.ds(j, _LANES)],
                    )

            out_copy(c, b).start()
            nxt = c + (_NBUF - 1)
            if nxt < n_chunks:
                nb = nxt % _NBUF
                if nxt >= _NBUF:
                    out_copy(nxt - _NBUF, nb).wait()
                gather(nxt, nb).start()

        for c in range(n_chunks - _NBUF, n_chunks):
            out_copy(c, c % _NBUF).wait()

    return emb_kernel(wte, labels_flat, wpe)


def _tc_passthrough(x, n_blocks=8):
    bs, s, d = x.shape
    blk = (bs, s // n_blocks, d)

    def body(x_ref, o_ref):
        o_ref[...] = x_ref[...]

    return pl.pallas_call(
        body,
        out_shape=jax.ShapeDtypeStruct(x.shape, x.dtype),
        grid=(n_blocks,),
        in_specs=[pl.BlockSpec(blk, lambda i: (0, i, 0))],
        out_specs=pl.BlockSpec(blk, lambda i: (0, i, 0)),
    )(x)


def kernel(hidden, labels, wte, wpe):
    b, s = labels.shape
    labels_flat = labels.reshape(b * s)
    emb = _emb_lookup(labels_flat, wte, wpe, b, s)
    hidden_out = _tc_passthrough(hidden)
    attention_mask = labels != 0
    return (hidden_out, emb, labels, attention_mask)


# speedup vs baseline: 1.0042x; 1.0042x over previous
import functools

import jax
import jax.numpy as jnp
from jax import lax
from jax.experimental import pallas as pl
from jax.experimental.pallas import tpu as pltpu
from jax.experimental.pallas import tpu_sc as plsc

_NUM_CORES = 2
_NUM_SUBCORES = 16
_NUM_WORKERS = _NUM_CORES * _NUM_SUBCORES
_LANES = 16
_NBUF = 6
_CHUNK = 8


def _emb_lookup(labels_flat, wte, wpe, batch, seq_len):
    n = labels_flat.shape[0]
    _, d = wte.shape
    pos_per_w = seq_len // _NUM_WORKERS
    per_w = n // _NUM_WORKERS
    n_chunks = per_w // _CHUNK
    chunks_per_seq = pos_per_w // _CHUNK

    mesh = plsc.VectorSubcoreMesh(core_axis_name="c", subcore_axis_name="s")

    @functools.partial(
        pl.kernel,
        mesh=mesh,
        out_type=jax.ShapeDtypeStruct((batch, seq_len, d), jnp.float32),
        scratch_types=(
            [pltpu.VMEM((per_w,), jnp.int32),
             pltpu.VMEM((pos_per_w, d), jnp.float32)]
            + [pltpu.VMEM((_CHUNK, d), jnp.float32) for _ in range(_NBUF)]
            + [pltpu.SemaphoreType.DMA for _ in range(2 * _NBUF)]
        ),
    )
    def emb_kernel(wte_hbm, idx_hbm, wpe_hbm, out_hbm, idx_v, wpe_v, *bufs_and_sems):
        rows = bufs_and_sems[0:_NBUF]
        isems = bufs_and_sems[_NBUF:2 * _NBUF]
        osems = bufs_and_sems[2 * _NBUF:3 * _NBUF]

        wid = lax.axis_index("s") * _NUM_CORES + lax.axis_index("c")
        pos_base = wid * pos_per_w

        wpe_load = pltpu.make_async_copy(
            wpe_hbm.at[pl.ds(pos_base, pos_per_w)], wpe_v, osems[0]
        )
        wpe_load.start()
        idx_loads = [
            pltpu.make_async_copy(
                idx_hbm.at[pl.ds(b_idx * seq_len + pos_base, pos_per_w)],
                idx_v.at[pl.ds(b_idx * pos_per_w, pos_per_w)],
                isems[0],
            )
            for b_idx in range(batch)
        ]
        for cp in idx_loads:
            cp.start()
        for cp in idx_loads:
            cp.wait()

        def gather(c, b):
            return pltpu.make_async_copy(
                wte_hbm.at[idx_v.at[pl.ds(c * _CHUNK, _CHUNK)]], rows[b], isems[b]
            )

        def out_copy(c, b):
            b_idx, cc = divmod(c, chunks_per_seq)
            return pltpu.make_async_copy(
                rows[b],
                out_hbm.at[b_idx, pl.ds(pos_base + cc * _CHUNK, _CHUNK)],
                osems[b],
            )

        for c0 in range(_NBUF - 1):
            gather(c0, c0).start()
        for c in range(n_chunks):
            b = c % _NBUF
            if c == 0:
                wpe_load.wait()
            gather(c, b).wait()
            wrow_base = (c % chunks_per_seq) * _CHUNK

            buf = rows[b]

            @plsc.parallel_loop(0, d, step=_LANES)
            def _lane_loop(j, _buf=buf, _wb=wrow_base):
                for r in range(_CHUNK):
                    plsc.addupdate(
                        _buf.at[r, pl.ds(j, _LANES)],
                        wpe_v[_wb + r, pl.ds(j, _LANES)],
                    )

            out_copy(c, b).start()
            nxt = c + (_NBUF - 1)
            if nxt < n_chunks:
                nb = nxt % _NBUF
                if nxt >= _NBUF:
                    out_copy(nxt - _NBUF, nb).wait()
                gather(nxt, nb).start()

        for c in range(n_chunks - _NBUF, n_chunks):
            out_copy(c, c % _NBUF).wait()

    return emb_kernel(wte, labels_flat, wpe)


def _tc_passthrough(x, n_blocks=8):
    bs, s, d = x.shape
    blk = (bs, s // n_blocks, d)

    def body(x_ref, o_ref):
        o_ref[...] = x_ref[...]

    return pl.pallas_call(
        body,
        out_shape=jax.ShapeDtypeStruct(x.shape, x.dtype),
        grid=(n_blocks,),
        in_specs=[pl.BlockSpec(blk, lambda i: (0, i, 0))],
        out_specs=pl.BlockSpec(blk, lambda i: (0, i, 0)),
    )(x)


def kernel(hidden, labels, wte, wpe):
    b, s = labels.shape
    labels_flat = labels.reshape(b * s)
    emb = _emb_lookup(labels_flat, wte, wpe, b, s)
    hidden_out = _tc_passthrough(hidden)
    attention_mask = labels != 0
    return (hidden_out, emb, labels, attention_mask)
